# Initial kernel scaffold; baseline (speedup 1.0000x reference)
import jax
import jax.numpy as jnp
from jax.experimental import pallas as pl


def kernel(x, edge_index, edge_type, batch, W1_root, W1_rel, b1, W2_root, W2_rel, b2, lin_W, lin_b):
    raise NotImplementedError("write your pallas kernel here")


# speedup vs baseline: 15.6033x; 15.6033x over previous
import functools

import jax
import jax.numpy as jnp
from jax import lax
from jax.experimental import pallas as pl
from jax.experimental.pallas import tpu as pltpu
from jax.experimental.pallas import tpu_sc as plsc

N = 10000
E = 320000
D = 128
R = 8
G = 64

NC = 2
NS = 16
NW = NC * NS

CH = 80
EPT = E // NW
EPC = E // NS
RPT = 624

_mesh = plsc.VectorSubcoreMesh(core_axis_name="c", subcore_axis_name="s")
_sc_params = pltpu.CompilerParams(needs_layout_passes=False)


def _zero_rows(rows_v, n_rows):
  zero = jnp.zeros((16,), jnp.float32)
  @pl.loop(0, n_rows)
  def _(i):
    for j in range(D // 16):
      rows_v[i, pl.ds(j * 16, 16)] = zero


@functools.partial(
    pl.kernel,
    out_type=jax.ShapeDtypeStruct((E,), jnp.float32),
    mesh=_mesh,
    compiler_params=_sc_params,
    scratch_types=[
        pltpu.VMEM((CH,), jnp.int32),
        pltpu.VMEM((CH,), jnp.float32),
        pltpu.VMEM((EPT,), jnp.int32),
        pltpu.VMEM((N * R,), jnp.float32),
        pltpu.VMEM((EPT,), jnp.float32),
        pltpu.VMEM_SHARED((N * R,), jnp.float32),
    ],
)
def _sc_scales(fdst_hbm, scale_hbm, idx_v, ones_v, fdst_loc, cnt_loc,
               scale_loc, cnt_sh):
  c = lax.axis_index("c")
  t = lax.axis_index("s")

  one = jnp.ones((16,), jnp.float32)
  for j in range(CH // 16):
    ones_v[pl.ds(j * 16, 16)] = one

  zb = N * R // NS
  zero = jnp.zeros((16,), jnp.float32)
  @pl.loop(0, CH // 16)
  def _(j):
    scale_loc[pl.ds(j * 16, 16)] = zero
  nfull = zb // CH
  rem = zb - nfull * CH
  @pl.loop(0, nfull)
  def _(k):
    pltpu.sync_copy(scale_loc.at[pl.ds(0, CH)], cnt_sh.at[pl.ds(t * zb + k * CH, CH)])
  if rem:
    pltpu.sync_copy(scale_loc.at[pl.ds(0, rem)], cnt_sh.at[pl.ds(t * zb + nfull * CH, rem)])
  plsc.subcore_barrier()

  @pl.loop(0, EPC // CH)
  def _(k):
    eb = t * EPC + k * CH
    pltpu.sync_copy(fdst_hbm.at[pl.ds(eb, CH)], idx_v)
    pltpu.sync_copy(ones_v, cnt_sh.at[idx_v], add=True)
  plsc.subcore_barrier()

  base = (c * NS + t) * EPT
  pltpu.sync_copy(fdst_hbm.at[pl.ds(base, EPT)], fdst_loc)
  pltpu.sync_copy(cnt_sh, cnt_loc)
  @pl.loop(0, EPT // 16)
  def _(i):
    idx = fdst_loc[pl.ds(i * 16, 16)]
    cv = plsc.load_gather(cnt_loc, [idx])
    scale_loc[pl.ds(i * 16, 16)] = 1.0 / jnp.maximum(cv, 1.0)
  pltpu.sync_copy(scale_loc, scale_hbm.at[pl.ds(base, EPT)])


@functools.partial(
    pl.kernel,
    out_type=jax.ShapeDtypeStruct((NC, N, D), jnp.float32),
    mesh=_mesh,
    compiler_params=_sc_params,
    scratch_types=[
        pltpu.VMEM((CH,), jnp.int32),
        pltpu.VMEM((CH,), jnp.int32),
        pltpu.VMEM((CH,), jnp.float32),
        pltpu.VMEM((CH, D), jnp.float32),
        pltpu.VMEM_SHARED((N, D), jnp.float32),
        pltpu.SemaphoreType.DMA,
    ],
)
def _sc_agg(y_hbm, gidx_hbm, scale_hbm, dst_hbm, out_hbm,
            idx_v, dst_v, scale_v, rows_v, acc_sh, sem):
  c = lax.axis_index("c")
  t = lax.axis_index("s")

  _zero_rows(rows_v, CH)
  r0 = t * RPT
  for k in range(RPT // CH):
    pltpu.sync_copy(rows_v, acc_sh.at[pl.ds(r0 + k * CH, CH)])
  rem = RPT - (RPT // CH) * CH
  pltpu.sync_copy(rows_v.at[pl.ds(0, rem)],
                  acc_sh.at[pl.ds(r0 + RPT - rem, rem)])
  @pl.when(t == NS - 1)
  def _():
    pltpu.sync_copy(rows_v.at[pl.ds(0, N - NS * RPT)],
                    acc_sh.at[pl.ds(NS * RPT, N - NS * RPT)])
  plsc.subcore_barrier()

  base = (c * NS + t) * EPT
  @pl.loop(0, EPT // CH)
  def _(k):
    eb = base + k * CH
    pltpu.sync_copy(gidx_hbm.at[pl.ds(eb, CH)], idx_v)
    pltpu.sync_copy(scale_hbm.at[pl.ds(eb, CH)], scale_v)
    pltpu.sync_copy(dst_hbm.at[pl.ds(eb, CH)], dst_v)
    pltpu.async_copy(y_hbm.at[idx_v], rows_v, sem).wait()
    @pl.loop(0, CH // 16)
    def _(g):
      sv = scale_v[pl.ds(g * 16, 16)]
      for lane in range(16):
        s = sv[lane]
        row = g * 16 + lane
        for j in range(D // 16):
          rows_v[row, pl.ds(j * 16, 16)] = rows_v[row, pl.ds(j * 16, 16)] * s
    pltpu.sync_copy(rows_v, acc_sh.at[dst_v], add=True)
  plsc.subcore_barrier()

  for k in range(RPT // CH):
    pltpu.sync_copy(acc_sh.at[pl.ds(r0 + k * CH, CH)], rows_v)
    pltpu.sync_copy(rows_v, out_hbm.at[c, pl.ds(r0 + k * CH, CH)])
  pltpu.sync_copy(acc_sh.at[pl.ds(r0 + RPT - rem, rem)],
                  rows_v.at[pl.ds(0, rem)])
  pltpu.sync_copy(rows_v.at[pl.ds(0, rem)],
                  out_hbm.at[c, pl.ds(r0 + RPT - rem, rem)])
  @pl.when(t == NS - 1)
  def _():
    pltpu.sync_copy(acc_sh.at[pl.ds(NS * RPT, N - NS * RPT)],
                    rows_v.at[pl.ds(0, N - NS * RPT)])
    pltpu.sync_copy(rows_v.at[pl.ds(0, N - NS * RPT)],
                    out_hbm.at[c, pl.ds(NS * RPT, N - NS * RPT)])


_TB = 2000


def _t1_body(h_ref, w_ref, b_ref, out0_ref, y_ref):
  res = jnp.dot(h_ref[...], w_ref[...], preferred_element_type=jnp.float32)
  out0_ref[...] = res[:, :D] + b_ref[...]
  y_ref[...] = res[:, D:].reshape(_TB * R, D)


def _t2_body(o_ref, p0_ref, p1_ref, w_ref, b_ref, out0_ref, y_ref):
  h = jax.nn.relu(o_ref[...] + p0_ref[...] + p1_ref[...])
  res = jnp.dot(h, w_ref[...], preferred_element_type=jnp.float32)
  out0_ref[...] = res[:, :D] + b_ref[...]
  y_ref[...] = res[:, D:].reshape(_TB * R, D)


def _t3_body(o_ref, p0_ref, p1_ref, batch_ref, lw_ref, lb_ref, out_ref):
  h = jax.nn.relu(o_ref[...] + p0_ref[...] + p1_ref[...])
  gid = lax.broadcasted_iota(jnp.int32, (N, G), 1)
  eq = (batch_ref[...] == gid).astype(jnp.float32)
  dn = (((0,), (0,)), ((), ()))
  gs = lax.dot_general(eq, h, dn, preferred_element_type=jnp.float32)
  ones = jnp.ones((N, D), jnp.float32)
  cnt = lax.dot_general(eq, ones, dn, preferred_element_type=jnp.float32)
  g = gs / jnp.maximum(cnt, 1.0)
  out_ref[...] = jnp.dot(g, lw_ref[...], preferred_element_type=jnp.float32) + lb_ref[...]


def _tc_layer1(h, wcat, b):
  grid = N // _TB
  return pl.pallas_call(
      _t1_body,
      grid=(grid,),
      in_specs=[
          pl.BlockSpec((_TB, D), lambda i: (i, 0)),
          pl.BlockSpec((D, D * (R + 1)), lambda i: (0, 0)),
          pl.BlockSpec((1, D), lambda i: (0, 0)),
      ],
      out_specs=[
          pl.BlockSpec((_TB, D), lambda i: (i, 0)),
          pl.BlockSpec((_TB * R, D), lambda i: (i, 0)),
      ],
      out_shape=[
          jax.ShapeDtypeStruct((N, D), jnp.float32),
          jax.ShapeDtypeStruct((N * R, D), jnp.float32),
      ],
  )(h, wcat, b)


def _tc_layer2(o, p0, p1, wcat, b):
  grid = N // _TB
  return pl.pallas_call(
      _t2_body,
      grid=(grid,),
      in_specs=[
          pl.BlockSpec((_TB, D), lambda i: (i, 0)),
          pl.BlockSpec((_TB, D), lambda i: (i, 0)),
          pl.BlockSpec((_TB, D), lambda i: (i, 0)),
          pl.BlockSpec((D, D * (R + 1)), lambda i: (0, 0)),
          pl.BlockSpec((1, D), lambda i: (0, 0)),
      ],
      out_specs=[
          pl.BlockSpec((_TB, D), lambda i: (i, 0)),
          pl.BlockSpec((_TB * R, D), lambda i: (i, 0)),
      ],
      out_shape=[
          jax.ShapeDtypeStruct((N, D), jnp.float32),
          jax.ShapeDtypeStruct((N * R, D), jnp.float32),
      ],
  )(o, p0, p1, wcat, b)


def _tc_final(o, p0, p1, batch2d, lw, lb):
  return pl.pallas_call(
      _t3_body,
      out_shape=jax.ShapeDtypeStruct((G, D), jnp.float32),
  )(o, p0, p1, batch2d, lw, lb)


def _wcat(w_root, w_rel):
  return jnp.concatenate(
      [w_root, w_rel.transpose(1, 0, 2).reshape(D, R * D)], axis=1)


def kernel(x, edge_index, edge_type, batch, W1_root, W1_rel, b1,
           W2_root, W2_rel, b2, lin_W, lin_b):
  src = edge_index[0].astype(jnp.int32)
  dst = edge_index[1].astype(jnp.int32)
  et = edge_type.astype(jnp.int32)
  gidx = src * R + et
  fdst = dst * R + et

  scale = _sc_scales(fdst)

  o1, y1 = _tc_layer1(x, _wcat(W1_root, W1_rel), b1.reshape(1, D))
  p1 = _sc_agg(y1, gidx, scale, dst)
  o2, y2 = _tc_layer2(o1, p1[0], p1[1], _wcat(W2_root, W2_rel),
                      b2.reshape(1, D))
  p2 = _sc_agg(y2, gidx, scale, dst)
  out = _tc_final(o2, p2[0], p2[1], batch.astype(jnp.int32).reshape(N, 1),
                  lin_W, lin_b.reshape(1, D))
  return out
